# Initial kernel scaffold; baseline (speedup 1.0000x reference)
import jax
import jax.numpy as jnp
from jax.experimental import pallas as pl


def kernel(x, edge_index, Wk, bk, Wq, bq, Wv, bv, Ws, bias):
    raise NotImplementedError("write your pallas kernel here")


# speedup vs baseline: 1.5738x; 1.5738x over previous
import functools

import jax
import jax.numpy as jnp
from jax import lax
from jax.experimental import pallas as pl
from jax.experimental.pallas import tpu as pltpu
from jax.experimental.pallas import tpu_sc as plsc

NC = 2
NS = 16
NW = NC * NS
CHUNK = 80


def _proj_body(x_ref, w_ref, b_ref, k_ref, qv_ref):
    cat = jnp.dot(x_ref[...], w_ref[...], preferred_element_type=jnp.float32)
    cat = cat + b_ref[...]
    k_ref[...] = cat[:, :128]
    qv_ref[...] = cat[:, 128:]


def _epilogue_body(p_ref, x_ref, w_ref, b_ref, o_ref):
    skip = jnp.dot(x_ref[...], w_ref[...], preferred_element_type=jnp.float32)
    o_ref[...] = p_ref[0] + p_ref[1] + skip + b_ref[...]


def _edge_body(src_hbm, dst_hbm, k_hbm, qv_hbm, out_hbm,
               src_v, dst_v, kd_v, qvs_v, msg_v, agg_sh, sem_g, sem_q):
    cid = lax.axis_index("c")
    sid = lax.axis_index("s")
    n_nodes = agg_sh.shape[0]
    rows_per_tile = n_nodes // NS

    zvec = jnp.zeros((16,), jnp.float32)

    @pl.loop(0, CHUNK)
    def _zero_rows(r):
        for s in range(8):
            msg_v[r, pl.ds(16 * s, 16)] = zvec

    nfull = rows_per_tile // CHUNK
    rem = rows_per_tile - nfull * CHUNK

    @pl.loop(0, nfull)
    def _zero_agg(i):
        pltpu.sync_copy(msg_v, agg_sh.at[pl.ds(sid * rows_per_tile + i * CHUNK, CHUNK)])

    if rem:
        pltpu.sync_copy(msg_v.at[pl.ds(0, rem)],
                        agg_sh.at[pl.ds(sid * rows_per_tile + nfull * CHUNK, rem)])

    plsc.subcore_barrier()

    e_total = src_hbm.shape[0]
    e_per_w = e_total // NW
    wid = cid * NS + sid
    base = wid * e_per_w

    @pl.loop(0, e_per_w // CHUNK)
    def _chunk(j):
        off = base + j * CHUNK
        pltpu.sync_copy(src_hbm.at[pl.ds(off, CHUNK)], src_v)
        pltpu.sync_copy(dst_hbm.at[pl.ds(off, CHUNK)], dst_v)
        gk = pltpu.async_copy(k_hbm.at[dst_v], kd_v, sem_g)
        gq = pltpu.async_copy(qv_hbm.at[src_v], qvs_v, sem_q)
        gk.wait()
        gq.wait()

        @pl.loop(0, CHUNK)
        def _row(r):
            for s in range(8):
                kk = kd_v[r, pl.ds(16 * s, 16)]
                qq = qvs_v[r, pl.ds(16 * s, 16)]
                vv = qvs_v[r, pl.ds(128 + 16 * s, 16)]
                gate = 1.0 / (1.0 + jnp.exp(-(kk + qq)))
                msg_v[r, pl.ds(16 * s, 16)] = gate * vv

        pltpu.sync_copy(msg_v, agg_sh.at[dst_v], add=True)

    plsc.subcore_barrier()

    pltpu.sync_copy(agg_sh.at[pl.ds(sid * rows_per_tile, rows_per_tile)],
                    out_hbm.at[cid, pl.ds(sid * rows_per_tile, rows_per_tile)])


def kernel(x, edge_index, Wk, bk, Wq, bq, Wv, bv, Ws, bias):
    n, d = x.shape
    e = edge_index.shape[1]
    src = edge_index[0]
    dst = edge_index[1]

    wcat = jnp.concatenate([Wk.T, Wq.T, Wv.T], axis=1)
    bcat = jnp.concatenate([bk, bq, bv])[None, :]
    blk = 1000
    grid = n // blk
    k_t, qv_t = pl.pallas_call(
        _proj_body,
        grid=(grid,),
        in_specs=[
            pl.BlockSpec((blk, d), lambda i: (i, 0)),
            pl.BlockSpec((d, 3 * d), lambda i: (0, 0)),
            pl.BlockSpec((1, 3 * d), lambda i: (0, 0)),
        ],
        out_specs=[
            pl.BlockSpec((blk, d), lambda i: (i, 0)),
            pl.BlockSpec((blk, 2 * d), lambda i: (i, 0)),
        ],
        out_shape=[
            jax.ShapeDtypeStruct((n, d), jnp.float32),
            jax.ShapeDtypeStruct((n, 2 * d), jnp.float32),
        ],
    )(x, wcat, bcat)

    mesh = plsc.VectorSubcoreMesh(core_axis_name="c", subcore_axis_name="s")
    partials = pl.kernel(
        _edge_body,
        out_type=jax.ShapeDtypeStruct((NC, n, d), jnp.float32),
        mesh=mesh,
        compiler_params=pltpu.CompilerParams(use_tc_tiling_on_sc=False),
        scratch_types=[
            pltpu.VMEM((CHUNK,), jnp.int32),
            pltpu.VMEM((CHUNK,), jnp.int32),
            pltpu.VMEM((CHUNK, d), jnp.float32),
            pltpu.VMEM((CHUNK, 2 * d), jnp.float32),
            pltpu.VMEM((CHUNK, d), jnp.float32),
            pltpu.VMEM_SHARED((n, d), jnp.float32),
            pltpu.SemaphoreType.DMA,
            pltpu.SemaphoreType.DMA,
        ],
    )(src, dst, k_t, qv_t)

    out = pl.pallas_call(
        _epilogue_body,
        grid=(grid,),
        in_specs=[
            pl.BlockSpec((NC, blk, d), lambda i: (0, i, 0)),
            pl.BlockSpec((blk, d), lambda i: (i, 0)),
            pl.BlockSpec((d, d), lambda i: (0, 0)),
            pl.BlockSpec((1, d), lambda i: (0, 0)),
        ],
        out_specs=pl.BlockSpec((blk, d), lambda i: (i, 0)),
        out_shape=jax.ShapeDtypeStruct((n, d), jnp.float32),
    )(partials, x, Ws.T, bias[None, :])
    return out


# speedup vs baseline: 1.7788x; 1.1303x over previous
import functools

import jax
import jax.numpy as jnp
from jax import lax
from jax.experimental import pallas as pl
from jax.experimental.pallas import tpu as pltpu
from jax.experimental.pallas import tpu_sc as plsc

NC = 2
NS = 16
NW = NC * NS
CHUNK = 40


def _proj_body(x_ref, w_ref, b_ref, k_ref, qv_ref):
    cat = jnp.dot(x_ref[...], w_ref[...], preferred_element_type=jnp.float32)
    cat = cat + b_ref[...]
    k_ref[...] = cat[:, :128]
    qv_ref[...] = cat[:, 128:]


def _epilogue_body(p_ref, x_ref, w_ref, b_ref, o_ref):
    skip = jnp.dot(x_ref[...], w_ref[...], preferred_element_type=jnp.float32)
    o_ref[...] = p_ref[0] + p_ref[1] + skip + b_ref[...]


def _edge_body(ei_hbm, k_hbm, qv_hbm, out_hbm,
               ei0, ei1, kd0, kd1, qvs0, qvs1, msg0, msg1, dstv0, dstv1,
               agg_sh, gsem0, gsem1, qsem0, qsem1, ssem0, ssem1):
    eiv = (ei0, ei1)
    kd = (kd0, kd1)
    qvs = (qvs0, qvs1)
    msg = (msg0, msg1)
    dstv = (dstv0, dstv1)
    gsem = (gsem0, gsem1)
    qsem = (qsem0, qsem1)
    ssem = (ssem0, ssem1)

    cid = lax.axis_index("c")
    sid = lax.axis_index("s")
    n_nodes = agg_sh.shape[0]
    rows_per_tile = n_nodes // NS
    d = k_hbm.shape[1]
    nc_chunks = ei_hbm.shape[1]
    wid = cid * NS + sid

    zvec = jnp.zeros((16,), jnp.float32)

    @pl.loop(0, CHUNK)
    def _zero_rows(r):
        for s in range(d // 16):
            msg0[r, pl.ds(16 * s, 16)] = zvec

    nfull = rows_per_tile // CHUNK
    rem = rows_per_tile - nfull * CHUNK

    @pl.loop(0, nfull)
    def _zero_agg(i):
        pltpu.sync_copy(msg0, agg_sh.at[pl.ds(sid * rows_per_tile + i * CHUNK, CHUNK)])

    if rem:
        pltpu.sync_copy(msg0.at[pl.ds(0, rem)],
                        agg_sh.at[pl.ds(sid * rows_per_tile + nfull * CHUNK, rem)])

    plsc.subcore_barrier()

    def issue_gather(j, b):
        pltpu.sync_copy(ei_hbm.at[wid, j], eiv[b])
        pltpu.async_copy(k_hbm.at[eiv[b].at[1]], kd[b], gsem[b])
        pltpu.async_copy(qv_hbm.at[eiv[b].at[0]], qvs[b], qsem[b])

    def chunk_iter(j, b, issue_next, wait_sc):
        nb = 1 - b
        if issue_next:
            issue_gather(j + 1, nb)
        pltpu.make_async_copy(k_hbm.at[eiv[b].at[1]], kd[b], gsem[b]).wait()
        pltpu.make_async_copy(qv_hbm.at[eiv[b].at[0]], qvs[b], qsem[b]).wait()
        if wait_sc:
            pltpu.make_async_copy(msg[b], agg_sh.at[dstv[b]], ssem[b]).wait()
        for o in (0, 16, CHUNK - 16):
            dstv[b][pl.ds(o, 16)] = eiv[b][1, pl.ds(o, 16)]

        @pl.loop(0, CHUNK)
        def _row(r):
            for s in range(d // 16):
                kk = kd[b][r, pl.ds(16 * s, 16)]
                qq = qvs[b][r, pl.ds(16 * s, 16)]
                vv = qvs[b][r, pl.ds(d + 16 * s, 16)]
                gate = 1.0 / (1.0 + jnp.exp(-(kk + qq)))
                msg[b][r, pl.ds(16 * s, 16)] = gate * vv

        pltpu.async_copy(msg[b], agg_sh.at[dstv[b]], ssem[b], add=True)

    issue_gather(0, 0)
    chunk_iter(0, 0, issue_next=True, wait_sc=False)
    chunk_iter(1, 1, issue_next=True, wait_sc=False)

    @pl.loop(1, nc_chunks // 2 - 1)
    def _main(jj):
        for h in range(2):
            chunk_iter(2 * jj + h, h, issue_next=True, wait_sc=True)

    chunk_iter(nc_chunks - 2, 0, issue_next=True, wait_sc=True)
    chunk_iter(nc_chunks - 1, 1, issue_next=False, wait_sc=True)

    pltpu.make_async_copy(msg[0], agg_sh.at[dstv[0]], ssem[0]).wait()
    pltpu.make_async_copy(msg[1], agg_sh.at[dstv[1]], ssem[1]).wait()

    plsc.subcore_barrier()

    pltpu.sync_copy(agg_sh.at[pl.ds(sid * rows_per_tile, rows_per_tile)],
                    out_hbm.at[cid, pl.ds(sid * rows_per_tile, rows_per_tile)])


def kernel(x, edge_index, Wk, bk, Wq, bq, Wv, bv, Ws, bias):
    n, d = x.shape
    e = edge_index.shape[1]
    nc_chunks = e // (NW * CHUNK)

    ei_r = edge_index.reshape(2, NW, nc_chunks, CHUNK).transpose(1, 2, 0, 3)

    wcat = jnp.concatenate([Wk.T, Wq.T, Wv.T], axis=1)
    bcat = jnp.concatenate([bk, bq, bv])[None, :]
    blk = 1000
    grid = n // blk
    k_t, qv_t = pl.pallas_call(
        _proj_body,
        grid=(grid,),
        in_specs=[
            pl.BlockSpec((blk, d), lambda i: (i, 0)),
            pl.BlockSpec((d, 3 * d), lambda i: (0, 0)),
            pl.BlockSpec((1, 3 * d), lambda i: (0, 0)),
        ],
        out_specs=[
            pl.BlockSpec((blk, d), lambda i: (i, 0)),
            pl.BlockSpec((blk, 2 * d), lambda i: (i, 0)),
        ],
        out_shape=[
            jax.ShapeDtypeStruct((n, d), jnp.float32),
            jax.ShapeDtypeStruct((n, 2 * d), jnp.float32),
        ],
    )(x, wcat, bcat)

    mesh = plsc.VectorSubcoreMesh(core_axis_name="c", subcore_axis_name="s")
    partials = pl.kernel(
        _edge_body,
        out_type=jax.ShapeDtypeStruct((NC, n, d), jnp.float32),
        mesh=mesh,
        compiler_params=pltpu.CompilerParams(use_tc_tiling_on_sc=False),
        scratch_types=[
            pltpu.VMEM((2, CHUNK), jnp.int32),
            pltpu.VMEM((2, CHUNK), jnp.int32),
            pltpu.VMEM((CHUNK, d), jnp.float32),
            pltpu.VMEM((CHUNK, d), jnp.float32),
            pltpu.VMEM((CHUNK, 2 * d), jnp.float32),
            pltpu.VMEM((CHUNK, 2 * d), jnp.float32),
            pltpu.VMEM((CHUNK, d), jnp.float32),
            pltpu.VMEM((CHUNK, d), jnp.float32),
            pltpu.VMEM((CHUNK,), jnp.int32),
            pltpu.VMEM((CHUNK,), jnp.int32),
            pltpu.VMEM_SHARED((n, d), jnp.float32),
            pltpu.SemaphoreType.DMA,
            pltpu.SemaphoreType.DMA,
            pltpu.SemaphoreType.DMA,
            pltpu.SemaphoreType.DMA,
            pltpu.SemaphoreType.DMA,
            pltpu.SemaphoreType.DMA,
        ],
    )(ei_r, k_t, qv_t)

    out = pl.pallas_call(
        _epilogue_body,
        grid=(grid,),
        in_specs=[
            pl.BlockSpec((NC, blk, d), lambda i: (0, i, 0)),
            pl.BlockSpec((blk, d), lambda i: (i, 0)),
            pl.BlockSpec((d, d), lambda i: (0, 0)),
            pl.BlockSpec((1, d), lambda i: (0, 0)),
        ],
        out_specs=pl.BlockSpec((blk, d), lambda i: (i, 0)),
        out_shape=jax.ShapeDtypeStruct((n, d), jnp.float32),
    )(partials, x, Ws.T, bias[None, :])
    return out


# speedup vs baseline: 6.7556x; 3.7977x over previous
import functools

import jax
import jax.numpy as jnp
from jax import lax
from jax.experimental import pallas as pl
from jax.experimental.pallas import tpu as pltpu
from jax.experimental.pallas import tpu_sc as plsc

NC = 2
NS = 16
NW = NC * NS
CHUNK = 40


def _proj_body(x_ref, w_ref, b_ref, k_ref, qv_ref):
    cat = jnp.dot(x_ref[...], w_ref[...], preferred_element_type=jnp.float32)
    cat = cat + b_ref[...]
    k_ref[...] = cat[:, :128]
    qv_ref[...] = cat[:, 128:]


def _epilogue_body(p_ref, x_ref, w_ref, b_ref, o_ref):
    skip = jnp.dot(x_ref[...], w_ref[...], preferred_element_type=jnp.float32)
    o_ref[...] = p_ref[0] + p_ref[1] + skip + b_ref[...]


def _edge_body(ei_hbm, k_hbm, qv_hbm, out_hbm,
               ei0, ei1, kd0, kd1, qvs0, qvs1, msg0, msg1, dstv0, dstv1,
               agg_sh, gsem0, gsem1, qsem0, qsem1, ssem0, ssem1):
    eiv = (ei0, ei1)
    kd = (kd0, kd1)
    qvs = (qvs0, qvs1)
    msg = (msg0, msg1)
    dstv = (dstv0, dstv1)
    gsem = (gsem0, gsem1)
    qsem = (qsem0, qsem1)
    ssem = (ssem0, ssem1)

    cid = lax.axis_index("c")
    sid = lax.axis_index("s")
    n_nodes = agg_sh.shape[0]
    rows_per_tile = n_nodes // NS
    d = k_hbm.shape[1]
    nc_chunks = ei_hbm.shape[1]
    wid = cid * NS + sid

    zvec = jnp.zeros((16,), jnp.float32)

    @pl.loop(0, CHUNK)
    def _zero_rows(r):
        for s in range(d // 16):
            msg0[r, pl.ds(16 * s, 16)] = zvec

    nfull = rows_per_tile // CHUNK
    rem = rows_per_tile - nfull * CHUNK

    @pl.loop(0, nfull)
    def _zero_agg(i):
        pltpu.sync_copy(msg0, agg_sh.at[pl.ds(sid * rows_per_tile + i * CHUNK, CHUNK)])

    if rem:
        pltpu.sync_copy(msg0.at[pl.ds(0, rem)],
                        agg_sh.at[pl.ds(sid * rows_per_tile + nfull * CHUNK, rem)])

    plsc.subcore_barrier()

    def issue_gather(j, b):
        pltpu.sync_copy(ei_hbm.at[wid, j], eiv[b])
        pltpu.async_copy(k_hbm.at[eiv[b].at[1]], kd[b], gsem[b])
        pltpu.async_copy(qv_hbm.at[eiv[b].at[0]], qvs[b], qsem[b])

    def chunk_iter(j, b, issue_next, wait_sc):
        nb = 1 - b
        if issue_next:
            issue_gather(j + 1, nb)
        pltpu.make_async_copy(k_hbm.at[eiv[b].at[1]], kd[b], gsem[b]).wait()
        pltpu.make_async_copy(qv_hbm.at[eiv[b].at[0]], qvs[b], qsem[b]).wait()
        if wait_sc:
            pltpu.make_async_copy(msg[b], agg_sh.at[dstv[b]], ssem[b]).wait()
        for o in (0, 16, CHUNK - 16):
            dstv[b][pl.ds(o, 16)] = eiv[b][1, pl.ds(o, 16)]

        ns = d // 16

        @pl.loop(0, CHUNK)
        def _row(r):
            kk = [kd[b][r, pl.ds(16 * s, 16)] for s in range(ns)]
            qq = [qvs[b][r, pl.ds(16 * s, 16)] for s in range(ns)]
            ss = [jnp.exp(-(kk[s] + qq[s])) for s in range(ns)]
            rr = [1.0 / (1.0 + ss[s]) for s in range(ns)]
            vv = [qvs[b][r, pl.ds(d + 16 * s, 16)] for s in range(ns)]
            for s in range(ns):
                msg[b][r, pl.ds(16 * s, 16)] = rr[s] * vv[s]

        pltpu.async_copy(msg[b], agg_sh.at[dstv[b]], ssem[b], add=True)

    issue_gather(0, 0)
    chunk_iter(0, 0, issue_next=True, wait_sc=False)
    chunk_iter(1, 1, issue_next=True, wait_sc=False)

    @pl.loop(1, nc_chunks // 2 - 1)
    def _main(jj):
        for h in range(2):
            chunk_iter(2 * jj + h, h, issue_next=True, wait_sc=True)

    chunk_iter(nc_chunks - 2, 0, issue_next=True, wait_sc=True)
    chunk_iter(nc_chunks - 1, 1, issue_next=False, wait_sc=True)

    pltpu.make_async_copy(msg[0], agg_sh.at[dstv[0]], ssem[0]).wait()
    pltpu.make_async_copy(msg[1], agg_sh.at[dstv[1]], ssem[1]).wait()

    plsc.subcore_barrier()

    pltpu.sync_copy(agg_sh.at[pl.ds(sid * rows_per_tile, rows_per_tile)],
                    out_hbm.at[cid, pl.ds(sid * rows_per_tile, rows_per_tile)])


def kernel(x, edge_index, Wk, bk, Wq, bq, Wv, bv, Ws, bias):
    n, d = x.shape
    e = edge_index.shape[1]
    nc_chunks = e // (NW * CHUNK)

    ei_r = edge_index.reshape(2, NW, nc_chunks, CHUNK).transpose(1, 2, 0, 3)

    wcat = jnp.concatenate([Wk.T, Wq.T, Wv.T], axis=1)
    bcat = jnp.concatenate([bk, bq, bv])[None, :]
    blk = 1000
    grid = n // blk
    k_t, qv_t = pl.pallas_call(
        _proj_body,
        grid=(grid,),
        in_specs=[
            pl.BlockSpec((blk, d), lambda i: (i, 0)),
            pl.BlockSpec((d, 3 * d), lambda i: (0, 0)),
            pl.BlockSpec((1, 3 * d), lambda i: (0, 0)),
        ],
        out_specs=[
            pl.BlockSpec((blk, d), lambda i: (i, 0)),
            pl.BlockSpec((blk, 2 * d), lambda i: (i, 0)),
        ],
        out_shape=[
            jax.ShapeDtypeStruct((n, d), jnp.float32),
            jax.ShapeDtypeStruct((n, 2 * d), jnp.float32),
        ],
    )(x, wcat, bcat)

    mesh = plsc.VectorSubcoreMesh(core_axis_name="c", subcore_axis_name="s")
    partials = pl.kernel(
        _edge_body,
        out_type=jax.ShapeDtypeStruct((NC, n, d), jnp.float32),
        mesh=mesh,
        compiler_params=pltpu.CompilerParams(use_tc_tiling_on_sc=False),
        scratch_types=[
            pltpu.VMEM((2, CHUNK), jnp.int32),
            pltpu.VMEM((2, CHUNK), jnp.int32),
            pltpu.VMEM((CHUNK, d), jnp.float32),
            pltpu.VMEM((CHUNK, d), jnp.float32),
            pltpu.VMEM((CHUNK, 2 * d), jnp.float32),
            pltpu.VMEM((CHUNK, 2 * d), jnp.float32),
            pltpu.VMEM((CHUNK, d), jnp.float32),
            pltpu.VMEM((CHUNK, d), jnp.float32),
            pltpu.VMEM((CHUNK,), jnp.int32),
            pltpu.VMEM((CHUNK,), jnp.int32),
            pltpu.VMEM_SHARED((n, d), jnp.float32),
            pltpu.SemaphoreType.DMA,
            pltpu.SemaphoreType.DMA,
            pltpu.SemaphoreType.DMA,
            pltpu.SemaphoreType.DMA,
            pltpu.SemaphoreType.DMA,
            pltpu.SemaphoreType.DMA,
        ],
    )(ei_r, k_t, qv_t)

    out = pl.pallas_call(
        _epilogue_body,
        grid=(grid,),
        in_specs=[
            pl.BlockSpec((NC, blk, d), lambda i: (0, i, 0)),
            pl.BlockSpec((blk, d), lambda i: (i, 0)),
            pl.BlockSpec((d, d), lambda i: (0, 0)),
            pl.BlockSpec((1, d), lambda i: (0, 0)),
        ],
        out_specs=pl.BlockSpec((blk, d), lambda i: (i, 0)),
        out_shape=jax.ShapeDtypeStruct((n, d), jnp.float32),
    )(partials, x, Ws.T, bias[None, :])
    return out
